# baseline (speedup 1.0000x reference)
import functools

import jax
import jax.numpy as jnp
from jax import lax
from jax.experimental import pallas as pl
from jax.experimental.pallas import tpu as pltpu
from jax.experimental.pallas import tpu_sc as plsc

_U = 100000
_I = 100000
_D = 32
_LAM = 0.001

_NNZ = 1600000
_K = 384
_NSUB = _K // 128
_CHUNKS = 264
_NNZ_PAD = 16 * _CHUNKS * _K
_NNZ_ALLOC = _NNZ_PAD + 3 * _K
_HALF = 50000
_ACC_ROWS = 50048
_ZROWS = 128
_TILE_ACC = _ACC_ROWS // 16
_TILE_OUT = 3128
_LAST_OUT = _HALF - 15 * _TILE_OUT


def _sc_push(table, edges, n_dst):
    mesh = plsc.VectorSubcoreMesh(core_axis_name="c", subcore_axis_name="s")

    @functools.partial(
        pl.kernel,
        out_type=jax.ShapeDtypeStruct((n_dst, _D), jnp.float32),
        mesh=mesh,
        scratch_types=[
            [pltpu.VMEM((2, _K), jnp.int32)] * 4,
            [pltpu.VMEM((_NSUB, 128), jnp.int32)] * 2,
            [pltpu.VMEM((_K, _D), jnp.float32)] * 2,
            [pltpu.SemaphoreType.DMA] * 4,
            [pltpu.SemaphoreType.DMA] * 2,
            [pltpu.SemaphoreType.DMA] * 2,
            pltpu.VMEM_SHARED((_ACC_ROWS, _D), jnp.float32),
        ],
        compiler_params=pltpu.CompilerParams(use_tc_tiling_on_sc=False),
    )
    def push(edges_hbm, table_hbm, out_hbm,
             ebuf, dst2d, rows, semE, semG, semS, acc):
        c = lax.axis_index("c")
        s = lax.axis_index("s")
        lo = c * _HALF
        tbase = s * (_CHUNKS * _K)

        def e_slice(off):
            return edges_hbm.at[:, pl.ds(off, _K)]

        pltpu.async_copy(e_slice(tbase), ebuf[0], semE[0])
        pltpu.async_copy(e_slice(tbase + _K), ebuf[1], semE[1])
        pltpu.async_copy(e_slice(tbase + 2 * _K), ebuf[2], semE[2])

        zero16 = jnp.zeros((16,), jnp.float32)

        @pl.loop(0, _K)
        def _(i):
            rows[0][i, pl.ds(0, 16)] = zero16
            rows[0][i, pl.ds(16, 16)] = zero16

        zbase = s * _TILE_ACC

        @pl.loop(0, _TILE_ACC // _K)
        def _(i):
            pltpu.sync_copy(rows[0], acc.at[pl.ds(zbase + i * _K, _K), :])

        _rem = _TILE_ACC % _K
        if _rem:
            pltpu.sync_copy(
                rows[0].at[pl.ds(0, _rem), :],
                acc.at[pl.ds(zbase + (_TILE_ACC // _K) * _K, _rem), :])

        plsc.subcore_barrier()

        lane = lax.iota(jnp.int32, 16)

        def gath(eb, b, j):
            return pltpu.make_async_copy(
                table_hbm.at[ebuf[eb].at[0, pl.ds(j * 128, 128)]],
                rows[b].at[pl.ds(j * 128, 128), :], semG[b])

        def scat(b, j):
            return pltpu.make_async_copy(
                rows[b].at[pl.ds(j * 128, 128), :],
                acc.at[dst2d[b].at[j]], semS[b])

        @pl.loop(0, _CHUNKS // 4)
        def _(t):
            g0 = 4 * t
            for p in range(4):
                b, ob, pe = p % 2, 1 - (p % 2), (p - 1) % 4
                pltpu.make_async_copy(e_slice(tbase), ebuf[p], semE[p]).wait()

                def waitS():
                    for j in range(_NSUB):
                        scat(b, j).wait()

                if p < 2:
                    pl.when(t >= 1)(waitS)
                else:
                    waitS()

                @pl.loop(0, _K // 16)
                def _(r):
                    d = ebuf[p][1, pl.ds(r * 16, 16)] - lo
                    ok = (d >= 0) & (d < _HALF)
                    dl = jnp.where(ok, d, _HALF + lane)
                    dst2d[b][r // 8, pl.ds((r % 8) * 16, 16)] = dl

                for j in range(_NSUB):
                    gath(p, b, j).start()

                def drain_prev():
                    for j in range(_NSUB):
                        gath(pe, ob, j).wait()
                    for j in range(_NSUB):
                        scat(ob, j).start(add=True)

                if p == 0:
                    pl.when(t >= 1)(drain_prev)
                else:
                    drain_prev()
                pltpu.async_copy(
                    e_slice(tbase + (g0 + p + 3) * _K), ebuf[pe], semE[pe])

        for j in range(_NSUB):
            gath(3, 1, j).wait()
        for j in range(_NSUB):
            scat(1, j).start(add=True)
        for j in range(_NSUB):
            scat(0, j).wait()
        for j in range(_NSUB):
            scat(1, j).wait()
        for p in range(3):
            pltpu.make_async_copy(e_slice(tbase), ebuf[p], semE[p]).wait()

        plsc.subcore_barrier()

        @pl.when(s < 15)
        def _():
            pltpu.sync_copy(
                acc.at[pl.ds(s * _TILE_OUT, _TILE_OUT), :],
                out_hbm.at[pl.ds(c * _HALF + s * _TILE_OUT, _TILE_OUT), :])

        @pl.when(s == 15)
        def _():
            pltpu.sync_copy(
                acc.at[pl.ds(15 * _TILE_OUT, _LAST_OUT), :],
                out_hbm.at[pl.ds(c * _HALF + 15 * _TILE_OUT, _LAST_OUT), :])

    return push(edges, table)


def kernel(user0, item_i0, ratings, edge_user, edge_item, edge_val, d_i, d_j,
           embed_user_w, embed_item_w, user_bias_w, item_bias_w, add_w,
           avg_rating):
    pad = _NNZ_ALLOC - _NNZ
    far = jnp.full((pad,), 1 << 29, dtype=jnp.int32)
    zpad = jnp.zeros((pad,), jnp.int32)
    ui_edges = jnp.stack([jnp.concatenate([edge_item, zpad]),
                          jnp.concatenate([edge_user, far])])
    iu_edges = jnp.stack([jnp.concatenate([edge_user, zpad]),
                          jnp.concatenate([edge_item, far])])

    su = jnp.sqrt(d_i)
    si = jnp.sqrt(d_j)

    Pu1 = _sc_push(si * embed_item_w, ui_edges, _U)
    Pi1 = _sc_push(su * embed_user_w, iu_edges, _I)
    gcn1_u = jax.nn.relu(su * Pu1 + embed_user_w * d_i)
    gcn1_i = jax.nn.relu(si * Pi1 + embed_item_w * d_j)
    Pu2 = _sc_push(si * gcn1_i, ui_edges, _U)
    Pi2 = _sc_push(su * gcn1_u, iu_edges, _I)
    gcn2_u = jax.nn.relu(su * Pu2 + gcn1_u * d_i)
    gcn2_i = jax.nn.relu(si * Pi2 + gcn1_i * d_j)

    w = add_w[0]
    gcn_u = embed_user_w * w[0] + gcn1_u * w[1] + gcn2_u * w[2]
    gcn_i = embed_item_w * w[0] + gcn1_i * w[1] + gcn2_i * w[2]

    user_bias = user_bias_w[user0][:, 0]
    item_bias = item_bias_w[item_i0][:, 0]
    user = gcn_u[user0]
    item_i = gcn_i[item_i0]

    prediction_i = (jnp.sum(user * item_i, axis=-1)
                    + user_bias + item_bias + avg_rating)
    l2 = _LAM * jnp.mean(gcn_u ** 2) + _LAM * jnp.mean(gcn_i ** 2)
    loss2 = jnp.mean((prediction_i - ratings) ** 2)
    loss = loss2 + l2
    return (loss, loss2, l2)


# speedup vs baseline: 1.4183x; 1.4183x over previous
import functools

import jax
import jax.numpy as jnp
from jax import lax
from jax.experimental import pallas as pl
from jax.experimental.pallas import tpu as pltpu
from jax.experimental.pallas import tpu_sc as plsc

_U = 100000
_I = 100000
_D = 32
_LAM = 0.001

_NNZ = 1600000
_K = 384
_NSUB = _K // 128
_CHUNKS = 262
_NNZ_PAD = 16 * _CHUNKS * _K
_NNZ_ALLOC = _NNZ_PAD + 2 * _K
_HALF = 50000
_ACC_ROWS = 51072
_ZROWS = 128
_TILE_ACC = _ACC_ROWS // 16
_TILE_OUT = 3128
_LAST_OUT = _HALF - 15 * _TILE_OUT


def _sc_push(table, edges, n_dst):
    mesh = plsc.VectorSubcoreMesh(core_axis_name="c", subcore_axis_name="s")

    @functools.partial(
        pl.kernel,
        out_type=jax.ShapeDtypeStruct((n_dst, _D), jnp.float32),
        mesh=mesh,
        scratch_types=[
            [pltpu.VMEM((2, _K), jnp.int32)] * 4,
            [pltpu.VMEM((_NSUB, 128), jnp.int32)] * 2,
            [pltpu.VMEM((_K, _D), jnp.float32)] * 2,
            [pltpu.SemaphoreType.DMA] * 4,
            [pltpu.SemaphoreType.DMA] * 2,
            [pltpu.SemaphoreType.DMA] * 2,
            pltpu.VMEM_SHARED((_ACC_ROWS, _D), jnp.float32),
        ],
        compiler_params=pltpu.CompilerParams(use_tc_tiling_on_sc=False),
    )
    def push(edges_hbm, table_hbm, out_hbm,
             ebuf, dst2d, rows, semE, semG, semS, acc):
        c = lax.axis_index("c")
        s = lax.axis_index("s")
        lo = c * _HALF
        tbase = s * (_CHUNKS * _K)

        def e_slice(off):
            return edges_hbm.at[:, pl.ds(off, _K)]

        pltpu.async_copy(e_slice(tbase), ebuf[0], semE[0])
        pltpu.async_copy(e_slice(tbase + _K), ebuf[1], semE[1])
        pltpu.async_copy(e_slice(tbase + 2 * _K), ebuf[2], semE[2])

        zero16 = jnp.zeros((16,), jnp.float32)

        @pl.loop(0, _K)
        def _(i):
            rows[0][i, pl.ds(0, 16)] = zero16
            rows[0][i, pl.ds(16, 16)] = zero16

        zbase = s * _TILE_ACC

        @pl.loop(0, _TILE_ACC // _K)
        def _(i):
            pltpu.sync_copy(rows[0], acc.at[pl.ds(zbase + i * _K, _K), :])

        _rem = _TILE_ACC % _K
        if _rem:
            pltpu.sync_copy(
                rows[0].at[pl.ds(0, _rem), :],
                acc.at[pl.ds(zbase + (_TILE_ACC // _K) * _K, _rem), :])

        plsc.subcore_barrier()

        lane = lax.iota(jnp.int32, 16)

        def gath(eb, b, j):
            return pltpu.make_async_copy(
                table_hbm.at[ebuf[eb].at[0, pl.ds(j * 128, 128)]],
                rows[b].at[pl.ds(j * 128, 128), :], semG[b])

        def scat(b, j):
            return pltpu.make_async_copy(
                rows[b].at[pl.ds(j * 128, 128), :],
                acc.at[dst2d[b].at[j]], semS[b])

        @pl.loop(0, _CHUNKS // 4)
        def _(t):
            g0 = 4 * t
            for p in range(4):
                b, ob, pe = p % 2, 1 - (p % 2), (p - 1) % 4
                pltpu.make_async_copy(e_slice(tbase), ebuf[p], semE[p]).wait()

                def waitS():
                    for j in range(_NSUB):
                        scat(b, j).wait()

                if p < 2:
                    pl.when(t >= 1)(waitS)
                else:
                    waitS()

                @pl.loop(0, _K // 16)
                def _(r):
                    d = ebuf[p][1, pl.ds(r * 16, 16)] - lo
                    ok = (d >= 0) & (d < _HALF)
                    dl = jnp.where(ok, d, _HALF + lane)
                    dst2d[b][r // 8, pl.ds((r % 8) * 16, 16)] = dl

                for j in range(_NSUB):
                    gath(p, b, j).start()

                def drain_prev():
                    for j in range(_NSUB):
                        gath(pe, ob, j).wait()
                    for j in range(_NSUB):
                        scat(ob, j).start(add=True)

                if p == 0:
                    pl.when(t >= 1)(drain_prev)
                else:
                    drain_prev()
                pltpu.async_copy(
                    e_slice(tbase + (g0 + p + 3) * _K), ebuf[pe], semE[pe])

        for j in range(_NSUB):
            gath(3, 1, j).wait()
        for j in range(_NSUB):
            scat(1, j).start(add=True)
        for j in range(_NSUB):
            scat(0, j).wait()
        for j in range(_NSUB):
            scat(1, j).wait()
        for p in range(3):
            pltpu.make_async_copy(e_slice(tbase), ebuf[p], semE[p]).wait()

        plsc.subcore_barrier()

        @pl.when(s < 15)
        def _():
            pltpu.sync_copy(
                acc.at[pl.ds(s * _TILE_OUT, _TILE_OUT), :],
                out_hbm.at[pl.ds(c * _HALF + s * _TILE_OUT, _TILE_OUT), :])

        @pl.when(s == 15)
        def _():
            pltpu.sync_copy(
                acc.at[pl.ds(15 * _TILE_OUT, _LAST_OUT), :],
                out_hbm.at[pl.ds(c * _HALF + 15 * _TILE_OUT, _LAST_OUT), :])

    return push(edges, table)


def kernel(user0, item_i0, ratings, edge_user, edge_item, edge_val, d_i, d_j,
           embed_user_w, embed_item_w, user_bias_w, item_bias_w, add_w,
           avg_rating):
    pad = _NNZ_ALLOC - _NNZ
    far = jnp.full((pad,), 1 << 29, dtype=jnp.int32)
    zpad = jnp.zeros((pad,), jnp.int32)
    ui_edges = jnp.stack([jnp.concatenate([edge_item, zpad]),
                          jnp.concatenate([edge_user, far])])
    iu_edges = jnp.stack([jnp.concatenate([edge_user, zpad]),
                          jnp.concatenate([edge_item, far])])

    su = jnp.sqrt(d_i)
    si = jnp.sqrt(d_j)

    Pu1 = _sc_push(si * embed_item_w, ui_edges, _U)
    Pi1 = _sc_push(su * embed_user_w, iu_edges, _I)
    gcn1_u = jax.nn.relu(su * Pu1 + embed_user_w * d_i)
    gcn1_i = jax.nn.relu(si * Pi1 + embed_item_w * d_j)
    Pu2 = _sc_push(si * gcn1_i, ui_edges, _U)
    Pi2 = _sc_push(su * gcn1_u, iu_edges, _I)
    gcn2_u = jax.nn.relu(su * Pu2 + gcn1_u * d_i)
    gcn2_i = jax.nn.relu(si * Pi2 + gcn1_i * d_j)

    w = add_w[0]
    gcn_u = embed_user_w * w[0] + gcn1_u * w[1] + gcn2_u * w[2]
    gcn_i = embed_item_w * w[0] + gcn1_i * w[1] + gcn2_i * w[2]

    user_bias = user_bias_w[user0][:, 0]
    item_bias = item_bias_w[item_i0][:, 0]
    user = gcn_u[user0]
    item_i = gcn_i[item_i0]

    prediction_i = (jnp.sum(user * item_i, axis=-1)
                    + user_bias + item_bias + avg_rating)
    l2 = _LAM * jnp.mean(gcn_u ** 2) + _LAM * jnp.mean(gcn_i ** 2)
    loss2 = jnp.mean((prediction_i - ratings) ** 2)
    loss = loss2 + l2
    return (loss, loss2, l2)
